# baseline (speedup 1.0000x reference)
import functools

import jax
import jax.numpy as jnp
from jax import lax
from jax.experimental import pallas as pl
from jax.experimental.pallas import tpu as pltpu
from jax.experimental.pallas import tpu_sc as plsc

N_SPEC = 4096
N_R1 = 65536
N_R2 = 262144
BATCH = 64

NGROUP = 4
NCHUNK = 8
K = 512


def _sc_partials(y2d, t16, i1r, p1, r1, i2ab, p2, r2):
    mesh = plsc.VectorSubcoreMesh(core_axis_name="c", subcore_axis_name="s")

    raw_set = [
        pltpu.VMEM((2 * K,), jnp.int32),
        pltpu.VMEM((K,), jnp.int32),
        pltpu.VMEM((K,), jnp.float32),
        pltpu.SemaphoreType.DMA,
    ]

    @functools.partial(
        pl.kernel,
        mesh=mesh,
        compiler_params=pltpu.CompilerParams(
            needs_layout_passes=False, use_tc_tiling_on_sc=False),
        out_type=jax.ShapeDtypeStruct((NGROUP, N_SPEC, 16), jnp.float32),
        scratch_types=[
            pltpu.VMEM((16,), jnp.float32),
            pltpu.VMEM((N_SPEC * 8,), jnp.int32),
            pltpu.VMEM((16, 512), jnp.float32),
            pltpu.VMEM((K,), jnp.int32),
            pltpu.VMEM((K,), jnp.int32),
            pltpu.VMEM((K,), jnp.float32),
            pltpu.VMEM((N_SPEC, 16), jnp.float32),
            pltpu.VMEM((32, 128), jnp.int32),
            pltpu.VMEM_SHARED((2 * N_SPEC, 16), jnp.float32),
        ] + raw_set + raw_set,
    )
    def k(y_hbm, t_hbm, i1r_hbm, p1_hbm, r1_hbm, i2ab_hbm, p2_hbm,
          r2_hbm, out_hbm, t_v, ytab, ybuf, ia8_v, ib8_v, rs_v, acc, idtab,
          shacc, pr0, p0, r0, sem0, pr1, p1_v, r1_v, sem1):
        core = lax.axis_index("c")
        sub = lax.axis_index("s")
        gl = sub % 2
        g = core * 2 + gl
        c = sub // 2

        sets = ((pr0, p0, r0, sem0), (pr1, p1_v, r1_v, sem1))

        pltpu.sync_copy(t_hbm, t_v)
        scale = jnp.exp(-t_v[...])
        iota = lax.iota(jnp.int32, 16)
        wsel = jnp.bitwise_and(iota, 7)
        shlv = jnp.where(iota < 8, 0, 16).astype(jnp.int32)
        pat_lo = jnp.bitwise_and(iota * 2, 15)
        pat_hi = pat_lo + 1
        lanes_lo = iota < 8

        for sl in range(8):
            pltpu.sync_copy(
                y_hbm.at[pl.ds(g * 16, 16), pl.ds(sl * 512, 512)], ybuf)

            @plsc.parallel_loop(0, 32, unroll=2)
            def pack(s16):
                sbase = s16 * 16
                svec8 = (iota + (sl * 512 + sbase)) * 8
                for w in range(8):
                    va = lax.bitcast_convert_type(
                        ybuf[w, pl.ds(sbase, 16)], jnp.int32)
                    vb = lax.bitcast_convert_type(
                        ybuf[w + 8, pl.ds(sbase, 16)], jnp.int32)
                    word = jnp.bitwise_or(
                        jnp.bitwise_and(va + 32768, jnp.int32(-65536)),
                        lax.shift_right_logical(vb + 32768, 16))
                    plsc.store_scatter(ytab, [svec8 + w], word)

        @plsc.parallel_loop(0, N_SPEC, unroll=8)
        def zero_body(i):
            acc[i, :] = jnp.zeros((16,), jnp.float32)

        for q in range(32):
            for j in range(8):
                idtab[q, pl.ds(j * 16, 16)] = (
                    iota + (gl * N_SPEC + q * 128 + j * 16))

        @pl.when(c == 0)
        def _():
            pltpu.sync_copy(acc, shacc.at[pl.ds(gl * N_SPEC, N_SPEC)])

        plsc.subcore_barrier()

        def fire(base, s, two_ops, ir_a, ir_p, ir_rate):
            pr_v, pv, rv, sem = sets[s]
            if two_ops:
                pltpu.async_copy(ir_a.at[pl.ds(2 * base, 2 * K)], pr_v, sem)
            else:
                pltpu.async_copy(ir_a.at[pl.ds(base, K)],
                                 pr_v.at[pl.ds(0, K)], sem)
            pltpu.async_copy(ir_p.at[pl.ds(base, K)], pv, sem)
            pltpu.async_copy(ir_rate.at[pl.ds(base, K)], rv, sem)

        def wait_fired(base, s, two_ops, ir_a, ir_p, ir_rate):
            pr_v, pv, rv, sem = sets[s]
            if two_ops:
                pltpu.make_async_copy(ir_a.at[pl.ds(2 * base, 2 * K)], pr_v,
                                      sem).wait()
            else:
                pltpu.make_async_copy(ir_a.at[pl.ds(base, K)],
                                      pr_v.at[pl.ds(0, K)], sem).wait()
            pltpu.make_async_copy(ir_p.at[pl.ds(base, K)], pv, sem).wait()
            pltpu.make_async_copy(ir_rate.at[pl.ds(base, K)], rv, sem).wait()

        def unpack(word):
            return lax.bitcast_convert_type(
                jnp.left_shift(word, shlv), jnp.float32)

        def prep_compute(s, two_ops):
            pr_v, pv, rv, _ = sets[s]

            @plsc.parallel_loop(0, K // 16, unroll=2)
            def pbody(j):
                sl = pl.ds(j * 16, 16)
                if two_ops:
                    v0 = pr_v[pl.ds(j * 32, 16)]
                    v1 = pr_v[pl.ds(j * 32 + 16, 16)]
                    ga0 = v0.at[pat_lo].get(mode="promise_in_bounds")
                    ga1 = v1.at[pat_lo].get(mode="promise_in_bounds")
                    gb0 = v0.at[pat_hi].get(mode="promise_in_bounds")
                    gb1 = v1.at[pat_hi].get(mode="promise_in_bounds")
                    ia8_v[sl] = jnp.where(lanes_lo, ga0, ga1) * 8
                    ib8_v[sl] = jnp.where(lanes_lo, gb0, gb1) * 8
                else:
                    ia8_v[sl] = pr_v[sl] * 8
                rs_v[sl] = rv[sl] * scale

            @plsc.parallel_loop(0, K // 16, unroll=2)
            def blk(b):
                bb = b * 16
                ia16 = ia8_v[pl.ds(bb, 16)]
                pf16 = pv[pl.ds(bb, 16)]
                rs16 = rs_v[pl.ds(bb, 16)]
                if two_ops:
                    ib16 = ib8_v[pl.ds(bb, 16)]

                wa, wb = {}, {}

                def load(kk):
                    idxa = jnp.broadcast_to(ia16[kk], (16,)) + wsel
                    wa[kk] = plsc.load_gather(ytab, [idxa])
                    if two_ops:
                        idxb = jnp.broadcast_to(ib16[kk], (16,)) + wsel
                        wb[kk] = plsc.load_gather(ytab, [idxb])

                load(0)
                load(1)
                load(2)
                for kk in range(16):
                    if kk + 3 < 16:
                        load(kk + 3)
                    va = unpack(wa[kk])
                    rk = jnp.broadcast_to(rs16[kk], (16,))
                    if two_ops:
                        term = va * unpack(wb[kk]) * rk
                    else:
                        term = va * rk
                    prow = jnp.broadcast_to(pf16[kk], (16,))
                    plsc.addupdate_scatter(acc, [prow, iota], term)

        def run_phase(nchunks, chunk_base, two_ops, ir_a, ir_p, ir_r):
            npair = nchunks // 2
            fire(chunk_base(0), 0, two_ops, ir_a, ir_p, ir_r)

            def pair(i, carry):
                wait_fired(chunk_base(2 * i), 0, two_ops, ir_a, ir_p, ir_r)
                fire(chunk_base(2 * i + 1), 1, two_ops, ir_a, ir_p, ir_r)
                prep_compute(0, two_ops)
                wait_fired(chunk_base(2 * i + 1), 1, two_ops, ir_a, ir_p,
                           ir_r)

                @pl.when(i < npair - 1)
                def _():
                    fire(chunk_base(2 * i + 2), 0, two_ops, ir_a, ir_p, ir_r)

                prep_compute(1, two_ops)
                return carry

            lax.fori_loop(0, npair, pair, 0)

        run_phase(N_R2 // NCHUNK // K,
                  lambda ci: c * (N_R2 // NCHUNK) + ci * K,
                  True, i2ab_hbm, p2_hbm, r2_hbm)
        run_phase(N_R1 // NCHUNK // K,
                  lambda ci: c * (N_R1 // NCHUNK) + ci * K,
                  False, i1r_hbm, p1_hbm, r1_hbm)

        for q in range(32):
            pltpu.sync_copy(acc.at[pl.ds(q * 128, 128)],
                            shacc.at[idtab.at[q]], add=True)

        plsc.subcore_barrier()

        pltpu.sync_copy(
            shacc.at[pl.ds(gl * N_SPEC + c * 512, 512)],
            out_hbm.at[g, pl.ds(c * 512, 512)])

    return k(y2d, t16, i1r, p1, r1, i2ab, p2, r2)


def kernel(t_in, y_in, inds_1r, inds_1p, rates_1, inds_2r, inds_2p, rates_2):
    t16 = jnp.broadcast_to(t_in.astype(jnp.float32), (16,))
    i1r = inds_1r.astype(jnp.int32)
    p1 = inds_1p.astype(jnp.int32)
    i2ab = inds_2r.astype(jnp.int32).reshape(-1)
    p2 = inds_2p.astype(jnp.int32)

    out = _sc_partials(y_in, t16, i1r, p1, rates_1, i2ab, p2, rates_2)
    return out.transpose(0, 2, 1).reshape(BATCH, N_SPEC)


# speedup vs baseline: 1.8963x; 1.8963x over previous
import functools

import jax
import jax.numpy as jnp
from jax import lax
from jax.experimental import pallas as pl
from jax.experimental.pallas import tpu as pltpu
from jax.experimental.pallas import tpu_sc as plsc

N_SPEC = 4096
N_R1 = 65536
N_R2 = 262144
BATCH = 64

NGROUP = 4
NCHUNK = 8
K = 512


def _sc_partials(y2d, t16, i1r, p1, r1, i2a, i2b, p2, r2):
    mesh = plsc.VectorSubcoreMesh(core_axis_name="c", subcore_axis_name="s")

    raw_set = [
        pltpu.VMEM((K,), jnp.int32),
        pltpu.VMEM((K,), jnp.int32),
        pltpu.VMEM((K,), jnp.int32),
        pltpu.VMEM((K,), jnp.float32),
        pltpu.SemaphoreType.DMA,
    ]

    @functools.partial(
        pl.kernel,
        mesh=mesh,
        compiler_params=pltpu.CompilerParams(
            needs_layout_passes=False, use_tc_tiling_on_sc=False),
        out_type=jax.ShapeDtypeStruct((NGROUP, N_SPEC, 16), jnp.float32),
        scratch_types=[
            pltpu.VMEM((16,), jnp.float32),
            pltpu.VMEM((N_SPEC * 8,), jnp.int32),
            pltpu.VMEM((16, 512), jnp.float32),
            pltpu.VMEM((K,), jnp.int32),
            pltpu.VMEM((K,), jnp.int32),
            pltpu.VMEM((K,), jnp.float32),
            pltpu.VMEM((N_SPEC, 16), jnp.float32),
            pltpu.VMEM((32, 128), jnp.int32),
            pltpu.VMEM_SHARED((2 * N_SPEC, 16), jnp.float32),
        ] + raw_set + raw_set,
    )
    def k(y_hbm, t_hbm, i1r_hbm, p1_hbm, r1_hbm, i2a_hbm, i2b_hbm, p2_hbm,
          r2_hbm, out_hbm, t_v, ytab, ybuf, ia8_v, ib8_v, rs_v, acc, idtab,
          shacc, ia0, ib0, p0, r0, sem0, ia1, ib1, p1_v, r1_v, sem1):
        core = lax.axis_index("c")
        sub = lax.axis_index("s")
        gl = sub % 2
        g = core * 2 + gl
        c = sub // 2

        sets = ((ia0, ib0, p0, r0, sem0), (ia1, ib1, p1_v, r1_v, sem1))

        pltpu.sync_copy(t_hbm, t_v)
        scale = jnp.exp(-t_v[...])
        iota = lax.iota(jnp.int32, 16)
        wsel = jnp.bitwise_and(iota, 7)
        shlv = jnp.where(iota < 8, 0, 16).astype(jnp.int32)

        for sl in range(8):
            pltpu.sync_copy(
                y_hbm.at[pl.ds(g * 16, 16), pl.ds(sl * 512, 512)], ybuf)

            @plsc.parallel_loop(0, 32, unroll=2)
            def pack(s16):
                sbase = s16 * 16
                svec8 = (iota + (sl * 512 + sbase)) * 8
                for w in range(8):
                    va = lax.bitcast_convert_type(
                        ybuf[w, pl.ds(sbase, 16)], jnp.int32)
                    vb = lax.bitcast_convert_type(
                        ybuf[w + 8, pl.ds(sbase, 16)], jnp.int32)
                    word = jnp.bitwise_or(
                        jnp.bitwise_and(va + 32768, jnp.int32(-65536)),
                        lax.shift_right_logical(vb + 32768, 16))
                    plsc.store_scatter(ytab, [svec8 + w], word)

        @plsc.parallel_loop(0, N_SPEC, unroll=8)
        def zero_body(i):
            acc[i, :] = jnp.zeros((16,), jnp.float32)

        for q in range(32):
            for j in range(8):
                idtab[q, pl.ds(j * 16, 16)] = (
                    iota + (gl * N_SPEC + q * 128 + j * 16))

        @pl.when(c == 0)
        def _():
            pltpu.sync_copy(acc, shacc.at[pl.ds(gl * N_SPEC, N_SPEC)])

        plsc.subcore_barrier()

        def fire(base, s, two_ops, ir_a, ir_b, ir_p, ir_rate):
            ia_v, ib_v, pv, rv, sem = sets[s]
            pltpu.async_copy(ir_a.at[pl.ds(base, K)], ia_v, sem)
            if two_ops:
                pltpu.async_copy(ir_b.at[pl.ds(base, K)], ib_v, sem)
            pltpu.async_copy(ir_p.at[pl.ds(base, K)], pv, sem)
            pltpu.async_copy(ir_rate.at[pl.ds(base, K)], rv, sem)

        def wait_fired(base, s, two_ops, ir_a, ir_b, ir_p, ir_rate):
            ia_v, ib_v, pv, rv, sem = sets[s]
            pltpu.make_async_copy(ir_a.at[pl.ds(base, K)], ia_v, sem).wait()
            if two_ops:
                pltpu.make_async_copy(ir_b.at[pl.ds(base, K)], ib_v,
                                      sem).wait()
            pltpu.make_async_copy(ir_p.at[pl.ds(base, K)], pv, sem).wait()
            pltpu.make_async_copy(ir_rate.at[pl.ds(base, K)], rv, sem).wait()

        def unpack(word):
            return lax.bitcast_convert_type(
                jnp.left_shift(word, shlv), jnp.float32)

        def prep_compute(s, two_ops):
            ia_v, ib_v, pv, rv, _ = sets[s]

            @plsc.parallel_loop(0, K // 16, unroll=2)
            def pbody(j):
                sl = pl.ds(j * 16, 16)
                ia8_v[sl] = ia_v[sl] * 8
                if two_ops:
                    ib8_v[sl] = ib_v[sl] * 8
                rs_v[sl] = rv[sl] * scale

            @plsc.parallel_loop(0, K // 16, unroll=2)
            def blk(b):
                bb = b * 16
                ia16 = ia8_v[pl.ds(bb, 16)]
                pf16 = pv[pl.ds(bb, 16)]
                rs16 = rs_v[pl.ds(bb, 16)]
                if two_ops:
                    ib16 = ib8_v[pl.ds(bb, 16)]

                wa, wb = {}, {}

                def load(kk):
                    idxa = jnp.broadcast_to(ia16[kk], (16,)) + wsel
                    wa[kk] = plsc.load_gather(ytab, [idxa])
                    if two_ops:
                        idxb = jnp.broadcast_to(ib16[kk], (16,)) + wsel
                        wb[kk] = plsc.load_gather(ytab, [idxb])

                load(0)
                load(1)
                load(2)
                for kk in range(16):
                    if kk + 3 < 16:
                        load(kk + 3)
                    va = unpack(wa[kk])
                    rk = jnp.broadcast_to(rs16[kk], (16,))
                    if two_ops:
                        term = va * unpack(wb[kk]) * rk
                    else:
                        term = va * rk
                    prow = jnp.broadcast_to(pf16[kk], (16,))
                    plsc.addupdate_scatter(acc, [prow, iota], term)

        def run_phase(nchunks, chunk_base, two_ops, ir_a, ir_b, ir_p, ir_r):
            npair = nchunks // 2
            fire(chunk_base(0), 0, two_ops, ir_a, ir_b, ir_p, ir_r)

            def pair(i, carry):
                wait_fired(chunk_base(2 * i), 0, two_ops, ir_a, ir_b, ir_p,
                           ir_r)
                fire(chunk_base(2 * i + 1), 1, two_ops, ir_a, ir_b, ir_p,
                     ir_r)
                prep_compute(0, two_ops)
                wait_fired(chunk_base(2 * i + 1), 1, two_ops, ir_a, ir_b,
                           ir_p, ir_r)

                @pl.when(i < npair - 1)
                def _():
                    fire(chunk_base(2 * i + 2), 0, two_ops, ir_a, ir_b, ir_p,
                         ir_r)

                prep_compute(1, two_ops)
                return carry

            lax.fori_loop(0, npair, pair, 0)

        run_phase(N_R2 // NCHUNK // K,
                  lambda ci: c * (N_R2 // NCHUNK) + ci * K,
                  True, i2a_hbm, i2b_hbm, p2_hbm, r2_hbm)
        run_phase(N_R1 // NCHUNK // K,
                  lambda ci: c * (N_R1 // NCHUNK) + ci * K,
                  False, i1r_hbm, i1r_hbm, p1_hbm, r1_hbm)

        for q in range(32):
            pltpu.sync_copy(acc.at[pl.ds(q * 128, 128)],
                            shacc.at[idtab.at[q]], add=True)

        plsc.subcore_barrier()

        pltpu.sync_copy(
            shacc.at[pl.ds(gl * N_SPEC + c * 512, 512)],
            out_hbm.at[g, pl.ds(c * 512, 512)])

    return k(y2d, t16, i1r, p1, r1, i2a, i2b, p2, r2)


def kernel(t_in, y_in, inds_1r, inds_1p, rates_1, inds_2r, inds_2p, rates_2):
    t16 = jnp.broadcast_to(t_in.astype(jnp.float32), (16,))
    i1r = inds_1r.astype(jnp.int32)
    p1 = inds_1p.astype(jnp.int32)
    i2a = inds_2r[:, 0].astype(jnp.int32)
    i2b = inds_2r[:, 1].astype(jnp.int32)
    p2 = inds_2p.astype(jnp.int32)

    out = _sc_partials(y_in, t16, i1r, p1, rates_1, i2a, i2b, p2, rates_2)
    return out.transpose(0, 2, 1).reshape(BATCH, N_SPEC)
